# baseline (speedup 1.0000x reference)
import functools

import jax
import jax.numpy as jnp
from jax import lax
from jax.experimental import pallas as pl
from jax.experimental.pallas import tpu as pltpu
from jax.experimental.pallas import tpu_sc as plsc

N = 90
NB = 96
E = 2700
HID = 256
LANES = 16
EVEC = 2688
NCHUNK = EVEC // LANES
NTAIL = E - EVEC


SU = 4


def _sc_count_body(adj_sc, adj_fc, out_sc, out_fc, edges_v, b_v):
    sid = lax.axis_index("s")

    def build(adj_hbm, out_hbm):
        pltpu.sync_copy(adj_hbm, edges_v)
        zeros = jnp.zeros((LANES,), jnp.float32)

        def zero_row(r, _):
            for j in range(NB // LANES):
                b_v[r, pl.ds(j * LANES, LANES)] = zeros
            return 0

        lax.fori_loop(0, N, zero_row, 0)

        def scat(i, _):
            for j in range(SU):
                base = (i * SU + j) * LANES
                src = edges_v[0, pl.ds(base, LANES)]
                dst = edges_v[1, pl.ds(base, LANES)]
                cnt, last = plsc.scan_count(dst * NB + src)
                plsc.addupdate_scatter(b_v, [dst, src],
                                       cnt.astype(jnp.float32), mask=last)
            return 0

        lax.fori_loop(0, NCHUNK // SU, scat, 0)
        src = edges_v[0, pl.ds(E - LANES, LANES)]
        dst = edges_v[1, pl.ds(E - LANES, LANES)]
        valid = lax.iota(jnp.int32, LANES) >= (LANES - NTAIL)
        cnt, last = plsc.scan_count(dst * NB + src, mask=valid)
        plsc.addupdate_scatter(b_v, [dst, src], cnt.astype(jnp.float32),
                               mask=last)
        pltpu.sync_copy(b_v, out_hbm)

    @pl.when(sid == 0)
    def _():
        build(adj_sc, out_sc)

    @pl.when(sid == 1)
    def _():
        build(adj_fc, out_fc)


_sc_count = functools.partial(
    pl.kernel,
    out_type=(
        jax.ShapeDtypeStruct((NB, NB), jnp.float32),
        jax.ShapeDtypeStruct((NB, NB), jnp.float32),
    ),
    mesh=plsc.VectorSubcoreMesh(core_axis_name="c", subcore_axis_name="s",
                                num_cores=1),
    compiler_params=pltpu.CompilerParams(needs_layout_passes=False,
                                         use_tc_tiling_on_sc=False),
    scratch_types=[
        pltpu.VMEM((2, E), jnp.int32),
        pltpu.VMEM((NB, NB), jnp.float32),
    ],
)(_sc_count_body)


def _gcn_layer(B, xw, b):
    deg = jnp.sum(B, axis=1, keepdims=True)
    dis = jnp.where(deg > 0, lax.rsqrt(jnp.maximum(deg, 1e-12)), 0.0)
    msg = dis * lax.dot(B, dis * xw)
    return jax.nn.relu(msg + b)


def _cai(h_sc, h_fc, Wb):
    P = lax.dot(h_sc, Wb)
    C = jnp.tanh(lax.dot_general(
        P, h_fc, (((1,), (1,)), ((), ()))))
    C_T = jnp.tanh(lax.dot_general(
        h_fc, P, (((1,), (1,)), ((), ()))))
    e1 = jnp.exp(C)
    e2 = jnp.exp(C_T)
    A_sc = e1 / jnp.sum(e1, axis=1, keepdims=True)
    A_fc = e2 / jnp.sum(e2, axis=1, keepdims=True)
    cosc = lax.dot(A_sc, h_fc)
    cofs = lax.dot(A_fc, h_sc)
    return cosc, cofs


def _fused_body(B_sc_ref, B_fc_ref, x_sc, x_fc, W0, b0, W1, b1, Wb,
                o1, o2, o3, o4):
    B_sc = B_sc_ref[...][:N, :N]
    B_fc = B_fc_ref[...][:N, :N]
    W0v, b0v = W0[...], jnp.reshape(b0[...], (1, HID))
    W1v, b1v = W1[...], jnp.reshape(b1[...], (1, HID))
    Wbv = Wb[...]
    h_sc = _gcn_layer(B_sc, lax.dot(x_sc[...], W0v), b0v)
    h_fc = _gcn_layer(B_fc, lax.dot(x_fc[...], W0v), b0v)
    cosc, cofs = _cai(h_sc, h_fc, Wbv)
    x_sc1 = jnp.concatenate([h_sc, cosc], axis=1)
    x_fc1 = jnp.concatenate([h_fc, cofs], axis=1)
    h_sc2 = _gcn_layer(B_sc, lax.dot(x_sc1, W1v), b1v)
    h_fc2 = _gcn_layer(B_fc, lax.dot(x_fc1, W1v), b1v)
    cosc2, cofs2 = _cai(h_sc2, h_fc2, Wbv)
    o1[...] = x_sc1
    o3[...] = x_fc1
    o2[...] = jnp.concatenate([h_sc2, cosc2], axis=1)
    o4[...] = jnp.concatenate([h_fc2, cofs2], axis=1)


@jax.jit
def kernel(x_sc, x_fc, adj_sc, adj_fc, W0, b0, W1, b1, Wb):
    B_sc, B_fc = _sc_count(adj_sc, adj_fc)
    out_sd = jax.ShapeDtypeStruct((N, 2 * HID), jnp.float32)
    r1, r2, r3, r4 = pl.pallas_call(
        _fused_body,
        out_shape=(out_sd, out_sd, out_sd, out_sd),
    )(B_sc, B_fc, x_sc, x_fc, W0, b0, W1, b1, Wb)
    return r1, r2, r3, r4


# speedup vs baseline: 1.0717x; 1.0717x over previous
import functools

import jax
import jax.numpy as jnp
from jax import lax
from jax.experimental import pallas as pl
from jax.experimental.pallas import tpu as pltpu
from jax.experimental.pallas import tpu_sc as plsc

N = 90
NB = 96
E = 2700
EP = 2816
HID = 256
LANES = 16
NCHUNK = EP // LANES


SU = 4


def _sc_count_body(adj_sc, adj_fc, out_sc, out_fc, edges_v, b_v):
    sid = lax.axis_index("s")

    def build(adj_hbm, out_hbm):
        pltpu.sync_copy(adj_hbm, edges_v)
        zeros = jnp.zeros((LANES,), jnp.float32)

        def zero_row(r, _):
            for j in range(NB // LANES):
                b_v[r, pl.ds(j * LANES, LANES)] = zeros
            return 0

        lax.fori_loop(0, N, zero_row, 0)

        def scat(i, _):
            for j in range(SU):
                base = (i * SU + j) * LANES
                src = edges_v[0, pl.ds(base, LANES)]
                dst = edges_v[1, pl.ds(base, LANES)]
                cnt, last = plsc.scan_count(dst * NB + src)
                plsc.addupdate_scatter(b_v, [dst, src],
                                       cnt.astype(jnp.float32), mask=last)
            return 0

        lax.fori_loop(0, NCHUNK // SU, scat, 0)
        pltpu.sync_copy(b_v, out_hbm)

    @pl.when(sid == 0)
    def _():
        build(adj_sc, out_sc)

    @pl.when(sid == 1)
    def _():
        build(adj_fc, out_fc)


_sc_count = functools.partial(
    pl.kernel,
    out_type=(
        jax.ShapeDtypeStruct((NB, NB), jnp.float32),
        jax.ShapeDtypeStruct((NB, NB), jnp.float32),
    ),
    mesh=plsc.VectorSubcoreMesh(core_axis_name="c", subcore_axis_name="s",
                                num_cores=1),
    compiler_params=pltpu.CompilerParams(needs_layout_passes=False),
    scratch_types=[
        pltpu.VMEM((2, EP), jnp.int32),
        pltpu.VMEM((NB, NB), jnp.float32),
    ],
)(_sc_count_body)


def _gcn_layer(B, xw, b):
    deg = jnp.sum(B, axis=1, keepdims=True)
    dis = jnp.where(deg > 0, lax.rsqrt(jnp.maximum(deg, 1e-12)), 0.0)
    msg = dis * lax.dot(B, dis * xw)
    return jax.nn.relu(msg + b)


def _cai(h_sc, h_fc, Wb):
    P = lax.dot(h_sc, Wb)
    C = jnp.tanh(lax.dot_general(
        P, h_fc, (((1,), (1,)), ((), ()))))
    C_T = jnp.tanh(lax.dot_general(
        h_fc, P, (((1,), (1,)), ((), ()))))
    e1 = jnp.exp(C)
    e2 = jnp.exp(C_T)
    A_sc = e1 / jnp.sum(e1, axis=1, keepdims=True)
    A_fc = e2 / jnp.sum(e2, axis=1, keepdims=True)
    cosc = lax.dot(A_sc, h_fc)
    cofs = lax.dot(A_fc, h_sc)
    return cosc, cofs


def _xw_body(x_sc, x_fc, W0, oxw_sc, oxw_fc):
    W0v = W0[...]
    oxw_sc[...] = lax.dot(x_sc[...], W0v)
    oxw_fc[...] = lax.dot(x_fc[...], W0v)


def _fused_body(B_sc_ref, B_fc_ref, xw_sc, xw_fc, b0, W1, b1, Wb,
                o1, o2, o3, o4):
    B_sc = B_sc_ref[...][:N, :N]
    B_fc = B_fc_ref[...][:N, :N]
    b0v = jnp.reshape(b0[...], (1, HID))
    W1v, b1v = W1[...], jnp.reshape(b1[...], (1, HID))
    Wbv = Wb[...]
    h_sc = _gcn_layer(B_sc, xw_sc[...], b0v)
    h_fc = _gcn_layer(B_fc, xw_fc[...], b0v)
    cosc, cofs = _cai(h_sc, h_fc, Wbv)
    x_sc1 = jnp.concatenate([h_sc, cosc], axis=1)
    x_fc1 = jnp.concatenate([h_fc, cofs], axis=1)
    h_sc2 = _gcn_layer(B_sc, lax.dot(x_sc1, W1v), b1v)
    h_fc2 = _gcn_layer(B_fc, lax.dot(x_fc1, W1v), b1v)
    cosc2, cofs2 = _cai(h_sc2, h_fc2, Wbv)
    o1[...] = x_sc1
    o3[...] = x_fc1
    o2[...] = jnp.concatenate([h_sc2, cosc2], axis=1)
    o4[...] = jnp.concatenate([h_fc2, cofs2], axis=1)


@jax.jit
def kernel(x_sc, x_fc, adj_sc, adj_fc, W0, b0, W1, b1, Wb):
    adj_sc_p = jnp.pad(adj_sc, ((0, 0), (0, EP - E)), constant_values=95)
    adj_fc_p = jnp.pad(adj_fc, ((0, 0), (0, EP - E)), constant_values=95)
    B_sc, B_fc = _sc_count(adj_sc_p, adj_fc_p)
    xw_sd = jax.ShapeDtypeStruct((N, HID), jnp.float32)
    xw_sc, xw_fc = pl.pallas_call(
        _xw_body, out_shape=(xw_sd, xw_sd),
    )(x_sc, x_fc, W0)
    out_sd = jax.ShapeDtypeStruct((N, 2 * HID), jnp.float32)
    r1, r2, r3, r4 = pl.pallas_call(
        _fused_body,
        out_shape=(out_sd, out_sd, out_sd, out_sd),
    )(B_sc, B_fc, xw_sc, xw_fc, b0, W1, b1, Wb)
    return r1, r2, r3, r4


# speedup vs baseline: 1.1553x; 1.0780x over previous
import functools

import jax
import jax.numpy as jnp
from jax import lax
from jax.experimental import pallas as pl
from jax.experimental.pallas import tpu as pltpu
from jax.experimental.pallas import tpu_sc as plsc

N = 90
NB = 96
E = 2700
EP = 2816
HID = 256
LANES = 16
NCHUNK = EP // LANES


SU = 4


NPART = 4
CPART = NCHUNK // NPART


def _sc_count_body(adj_sc, adj_fc, out_sc, out_fc, edges_v, b_v):
    sid = lax.axis_index("s")
    part = sid % NPART

    def build(adj_hbm, out_hbm):
        pltpu.sync_copy(adj_hbm, edges_v)
        zeros = jnp.zeros((LANES,), jnp.float32)

        def zero_row(r, _):
            for j in range(NB // LANES):
                b_v[r, pl.ds(j * LANES, LANES)] = zeros
            return 0

        lax.fori_loop(0, NB, zero_row, 0)

        def scat(i, _):
            for j in range(SU):
                base = (part * CPART + i * SU + j) * LANES
                src = edges_v[0, pl.ds(base, LANES)]
                dst = edges_v[1, pl.ds(base, LANES)]
                cnt, last = plsc.scan_count(dst * NB + src)
                plsc.addupdate_scatter(b_v, [dst, src],
                                       cnt.astype(jnp.float32), mask=last)
            return 0

        lax.fori_loop(0, CPART // SU, scat, 0)
        pltpu.sync_copy(b_v, out_hbm.at[part])

    @pl.when(sid < NPART)
    def _():
        build(adj_sc, out_sc)

    @pl.when(jnp.logical_and(sid >= NPART, sid < 2 * NPART))
    def _():
        build(adj_fc, out_fc)


_sc_count = functools.partial(
    pl.kernel,
    out_type=(
        jax.ShapeDtypeStruct((NPART, NB, NB), jnp.float32),
        jax.ShapeDtypeStruct((NPART, NB, NB), jnp.float32),
    ),
    mesh=plsc.VectorSubcoreMesh(core_axis_name="c", subcore_axis_name="s",
                                num_cores=1),
    compiler_params=pltpu.CompilerParams(needs_layout_passes=False),
    scratch_types=[
        pltpu.VMEM((2, EP), jnp.int32),
        pltpu.VMEM((NB, NB), jnp.float32),
    ],
)(_sc_count_body)


def _gcn_layer(B, xw, b):
    deg = jnp.sum(B, axis=1, keepdims=True)
    dis = jnp.where(deg > 0, lax.rsqrt(jnp.maximum(deg, 1e-12)), 0.0)
    msg = dis * lax.dot(B, dis * xw)
    return jax.nn.relu(msg + b)


def _cai(h_sc, h_fc, Wb):
    P = lax.dot(h_sc, Wb)
    C = jnp.tanh(lax.dot_general(
        P, h_fc, (((1,), (1,)), ((), ()))))
    C_T = jnp.tanh(lax.dot_general(
        h_fc, P, (((1,), (1,)), ((), ()))))
    e1 = jnp.exp(C)
    e2 = jnp.exp(C_T)
    A_sc = e1 / jnp.sum(e1, axis=1, keepdims=True)
    A_fc = e2 / jnp.sum(e2, axis=1, keepdims=True)
    cosc = lax.dot(A_sc, h_fc)
    cofs = lax.dot(A_fc, h_sc)
    return cosc, cofs


def _xw_body(x_sc, x_fc, W0, oxw_sc, oxw_fc):
    W0v = W0[...]
    oxw_sc[...] = lax.dot(x_sc[...], W0v)
    oxw_fc[...] = lax.dot(x_fc[...], W0v)


def _fused_body(B_sc_ref, B_fc_ref, xw_sc, xw_fc, b0, W1, b1, Wb,
                o1, o2, o3, o4):
    B_sc = jnp.sum(B_sc_ref[...], axis=0)[:N, :N]
    B_fc = jnp.sum(B_fc_ref[...], axis=0)[:N, :N]
    b0v = jnp.reshape(b0[...], (1, HID))
    W1v, b1v = W1[...], jnp.reshape(b1[...], (1, HID))
    Wbv = Wb[...]
    h_sc = _gcn_layer(B_sc, xw_sc[...], b0v)
    h_fc = _gcn_layer(B_fc, xw_fc[...], b0v)
    cosc, cofs = _cai(h_sc, h_fc, Wbv)
    x_sc1 = jnp.concatenate([h_sc, cosc], axis=1)
    x_fc1 = jnp.concatenate([h_fc, cofs], axis=1)
    h_sc2 = _gcn_layer(B_sc, lax.dot(x_sc1, W1v), b1v)
    h_fc2 = _gcn_layer(B_fc, lax.dot(x_fc1, W1v), b1v)
    cosc2, cofs2 = _cai(h_sc2, h_fc2, Wbv)
    o1[...] = x_sc1
    o3[...] = x_fc1
    o2[...] = jnp.concatenate([h_sc2, cosc2], axis=1)
    o4[...] = jnp.concatenate([h_fc2, cofs2], axis=1)


@jax.jit
def kernel(x_sc, x_fc, adj_sc, adj_fc, W0, b0, W1, b1, Wb):
    adj_sc_p = jnp.pad(adj_sc, ((0, 0), (0, EP - E)), constant_values=95)
    adj_fc_p = jnp.pad(adj_fc, ((0, 0), (0, EP - E)), constant_values=95)
    B_sc, B_fc = _sc_count(adj_sc_p, adj_fc_p)
    xw_sd = jax.ShapeDtypeStruct((N, HID), jnp.float32)
    xw_sc, xw_fc = pl.pallas_call(
        _xw_body, out_shape=(xw_sd, xw_sd),
    )(x_sc, x_fc, W0)
    out_sd = jax.ShapeDtypeStruct((N, 2 * HID), jnp.float32)
    r1, r2, r3, r4 = pl.pallas_call(
        _fused_body,
        out_shape=(out_sd, out_sd, out_sd, out_sd),
    )(B_sc, B_fc, xw_sc, xw_fc, b0, W1, b1, Wb)
    return r1, r2, r3, r4
